# Initial kernel scaffold; baseline (speedup 1.0000x reference)
import jax
import jax.numpy as jnp
from jax.experimental import pallas as pl


def kernel(x, edge_index, W, b, gamma, beta):
    raise NotImplementedError("write your pallas kernel here")


# speedup vs baseline: 27.9134x; 27.9134x over previous
import functools

import jax
import jax.numpy as jnp
from jax import lax
from jax.experimental import pallas as pl
from jax.experimental.pallas import tpu as pltpu
from jax.experimental.pallas import tpu_sc as plsc

N = 10000
E = 320000
D = 128
BN_EPS = 1e-5

NC = 2
NS = 16
NW = NC * NS
EPW = E // NW
CH = 80
NCHUNK = EPW // CH
R0 = (N // NS) // 8 * 8
RTAIL = N - NS * R0

_sc_mesh = plsc.VectorSubcoreMesh(
    core_axis_name="c", subcore_axis_name="s", num_cores=NC, num_subcores=NS
)


def _deg_body(dst_hbm, out_hbm, idx_v, ones_v, zb_v, sdeg):
    c = lax.axis_index("c")
    s = lax.axis_index("s")
    w = c * NS + s
    pltpu.sync_copy(dst_hbm.at[w], idx_v)

    def fill(i, carry):
        ones_v[pl.ds(i * 16, 16)] = jnp.ones((16,), jnp.float32)
        return carry

    lax.fori_loop(0, CH // 16, fill, 0)

    @pl.when(s == 0)
    def _zero():
        def z(i, carry):
            zb_v[pl.ds(i * 16, 16)] = jnp.zeros((16,), jnp.float32)
            return carry

        lax.fori_loop(0, N // 16, z, 0)
        pltpu.sync_copy(zb_v, sdeg)

    plsc.subcore_barrier()

    def acc(j, carry):
        pltpu.sync_copy(ones_v, sdeg.at[idx_v.at[j]], add=True)
        return carry

    lax.fori_loop(0, NCHUNK, acc, 0)
    plsc.subcore_barrier()

    @pl.when(s == 0)
    def _out():
        pltpu.sync_copy(sdeg, out_hbm.at[c])


_deg_call = pl.kernel(
    _deg_body,
    out_type=jax.ShapeDtypeStruct((NC, N), jnp.float32),
    mesh=_sc_mesh,
    scratch_types=[
        pltpu.VMEM((NCHUNK, CH), jnp.int32),
        pltpu.VMEM((CH,), jnp.float32),
        pltpu.VMEM((N,), jnp.float32),
        pltpu.VMEM_SHARED((N,), jnp.float32),
    ],
)


def _agg_body(hp_hbm, src_hbm, dst_hbm, out_hbm, sidx, didx, buf, acc_sh, sem):
    c = lax.axis_index("c")
    s = lax.axis_index("s")
    w = c * NS + s
    pltpu.sync_copy(src_hbm.at[w], sidx)
    pltpu.sync_copy(dst_hbm.at[w], didx)
    base = pl.multiple_of(s * R0, 8)
    pltpu.sync_copy(hp_hbm.at[pl.ds(base, R0)], acc_sh.at[pl.ds(base, R0)])

    @pl.when(s == NS - 1)
    def _init_tail():
        pltpu.sync_copy(hp_hbm.at[pl.ds(NS * R0, RTAIL)], acc_sh.at[pl.ds(NS * R0, RTAIL)])

    plsc.subcore_barrier()

    def step(j, carry):
        pltpu.async_copy(hp_hbm.at[sidx.at[j]], buf, sem).wait()
        pltpu.sync_copy(buf, acc_sh.at[didx.at[j]], add=True)
        return carry

    lax.fori_loop(0, NCHUNK, step, 0)
    plsc.subcore_barrier()
    pltpu.sync_copy(acc_sh.at[pl.ds(base, R0)], out_hbm.at[c].at[pl.ds(base, R0)])

    @pl.when(s == NS - 1)
    def _out_tail():
        pltpu.sync_copy(
            acc_sh.at[pl.ds(NS * R0, RTAIL)], out_hbm.at[c].at[pl.ds(NS * R0, RTAIL)]
        )


_agg_call = pl.kernel(
    _agg_body,
    out_type=jax.ShapeDtypeStruct((NC, N, D), jnp.float32),
    mesh=_sc_mesh,
    scratch_types=[
        pltpu.VMEM((NCHUNK, CH), jnp.int32),
        pltpu.VMEM((NCHUNK, CH), jnp.int32),
        pltpu.VMEM((CH, D), jnp.float32),
        pltpu.VMEM_SHARED((N, D), jnp.float32),
        pltpu.SemaphoreType.DMA,
    ],
)


def _mm_body(x_ref, w_ref, degt_ref, hp_ref):
    deg = degt_ref[:, 0:1] + degt_ref[:, 1:2] + 1.0
    dinv = lax.rsqrt(jnp.maximum(deg, 1.0))
    h = jnp.dot(x_ref[...], w_ref[...], preferred_element_type=jnp.float32)
    hp_ref[...] = h * dinv


_mm_call = pl.pallas_call(
    _mm_body,
    out_shape=jax.ShapeDtypeStruct((N, D), jnp.float32),
)


def _bn_body(s_ref, hp_ref, degt_ref, b_ref, g_ref, be_ref, z_ref):
    deg = degt_ref[:, 0:1] + degt_ref[:, 1:2] + 1.0
    dinv = lax.rsqrt(jnp.maximum(deg, 1.0))
    t = (s_ref[0] + s_ref[1] - hp_ref[...]) * dinv + b_ref[...]
    r = jnp.maximum(t, 0.0)
    mean = jnp.mean(r, axis=0, keepdims=True)
    cent = r - mean
    var = jnp.mean(cent * cent, axis=0, keepdims=True)
    z_ref[...] = cent * lax.rsqrt(var + BN_EPS) * g_ref[...] + be_ref[...]


_bn_call = pl.pallas_call(
    _bn_body,
    out_shape=jax.ShapeDtypeStruct((N, D), jnp.float32),
)


@jax.jit
def kernel(x, edge_index, W, b, gamma, beta):
    src = edge_index[0].reshape(NW, NCHUNK, CH)
    dst = edge_index[1].reshape(NW, NCHUNK, CH)
    degp = _deg_call(dst)
    degt = degp.T
    hp = _mm_call(x, W, degt)
    s_parts = _agg_call(hp, src, dst)
    z = _bn_call(
        s_parts, hp, degt,
        b.reshape(1, D), gamma.reshape(1, D), beta.reshape(1, D),
    )
    return z


# speedup vs baseline: 40.7819x; 1.4610x over previous
import functools

import jax
import jax.numpy as jnp
from jax import lax
from jax.experimental import pallas as pl
from jax.experimental.pallas import tpu as pltpu
from jax.experimental.pallas import tpu_sc as plsc

N = 10000
E = 320000
D = 128
BN_EPS = 1e-5

NC = 2
NS = 16
NW = NC * NS
EPW = E // NW
CH = 80
NCHUNK = EPW // CH
R0 = (N // NS) // 8 * 8
RTAIL = N - NS * R0

_sc_mesh = plsc.VectorSubcoreMesh(
    core_axis_name="c", subcore_axis_name="s", num_cores=NC, num_subcores=NS
)


def _deg_body(dst_hbm, zeros_hbm, out_hbm, idx_v, ones_v, sdeg):
    c = lax.axis_index("c")
    s = lax.axis_index("s")
    w = c * NS + s
    pltpu.sync_copy(dst_hbm.at[w], idx_v)

    for off in list(range(0, CH - 15, 16)) + [CH - 16]:
        ones_v[pl.ds(off, 16)] = jnp.ones((16,), jnp.float32)

    @pl.when(s == 0)
    def _zero():
        pltpu.sync_copy(zeros_hbm, sdeg)

    plsc.subcore_barrier()

    def acc(j, carry):
        pltpu.sync_copy(ones_v, sdeg.at[idx_v.at[j]], add=True)
        return carry

    lax.fori_loop(0, NCHUNK, acc, 0)
    plsc.subcore_barrier()

    @pl.when(s == 0)
    def _out():
        pltpu.sync_copy(sdeg, out_hbm.at[c])


_deg_call = pl.kernel(
    _deg_body,
    out_type=jax.ShapeDtypeStruct((NC, N), jnp.float32),
    mesh=_sc_mesh,
    scratch_types=[
        pltpu.VMEM((NCHUNK, CH), jnp.int32),
        pltpu.VMEM((CH,), jnp.float32),
        pltpu.VMEM_SHARED((N,), jnp.float32),
    ],
)


def _agg_body(hp_hbm, src_hbm, dst_hbm, out_hbm, sidx, didx, buf0, buf1, acc_sh, sem0, sem1):
    c = lax.axis_index("c")
    s = lax.axis_index("s")
    w = c * NS + s
    pltpu.sync_copy(src_hbm.at[pl.ds(pl.multiple_of(w * EPW, 8), EPW)], sidx)
    pltpu.sync_copy(dst_hbm.at[w], didx)
    base = pl.multiple_of(s * R0, 8)
    pltpu.sync_copy(hp_hbm.at[pl.ds(base, R0)], acc_sh.at[pl.ds(base, R0)])

    @pl.when(s == NS - 1)
    def _init_tail():
        pltpu.sync_copy(hp_hbm.at[pl.ds(NS * R0, RTAIL)], acc_sh.at[pl.ds(NS * R0, RTAIL)])

    plsc.subcore_barrier()

    def gather(j, buf, sem):
        pltpu.async_copy(hp_hbm.at[sidx.at[pl.ds(j * CH, CH)]], buf, sem)

    def drain_scatter(j, buf, sem):
        pltpu.make_async_copy(hp_hbm.at[sidx.at[pl.ds(j * CH, CH)]], buf, sem).wait()
        pltpu.sync_copy(buf, acc_sh.at[didx.at[j]], add=True)

    gather(0, buf0, sem0)

    def step(i, carry):
        j = 2 * i
        gather(j + 1, buf1, sem1)
        drain_scatter(j, buf0, sem0)

        @pl.when(j + 2 < NCHUNK)
        def _next():
            gather(j + 2, buf0, sem0)

        drain_scatter(j + 1, buf1, sem1)
        return carry

    lax.fori_loop(0, NCHUNK // 2, step, 0)
    if NCHUNK % 2:
        drain_scatter(NCHUNK - 1, buf0, sem0)
    plsc.subcore_barrier()
    pltpu.sync_copy(acc_sh.at[pl.ds(base, R0)], out_hbm.at[c].at[pl.ds(base, R0)])

    @pl.when(s == NS - 1)
    def _out_tail():
        pltpu.sync_copy(
            acc_sh.at[pl.ds(NS * R0, RTAIL)], out_hbm.at[c].at[pl.ds(NS * R0, RTAIL)]
        )


_agg_call = pl.kernel(
    _agg_body,
    out_type=jax.ShapeDtypeStruct((NC, N, D), jnp.float32),
    mesh=_sc_mesh,
    scratch_types=[
        pltpu.VMEM((EPW,), jnp.int32),
        pltpu.VMEM((NCHUNK, CH), jnp.int32),
        pltpu.VMEM((CH, D), jnp.float32),
        pltpu.VMEM((CH, D), jnp.float32),
        pltpu.VMEM_SHARED((N, D), jnp.float32),
        pltpu.SemaphoreType.DMA,
        pltpu.SemaphoreType.DMA,
    ],
)


def _mm_body(x_ref, w_ref, degt_ref, hp_ref):
    deg = degt_ref[:, 0:1] + degt_ref[:, 1:2] + 1.0
    dinv = lax.rsqrt(jnp.maximum(deg, 1.0))
    h = jnp.dot(x_ref[...], w_ref[...], preferred_element_type=jnp.float32)
    hp_ref[...] = h * dinv


_mm_call = pl.pallas_call(
    _mm_body,
    out_shape=jax.ShapeDtypeStruct((N, D), jnp.float32),
)


def _bn_body(s_ref, hp_ref, degt_ref, b_ref, g_ref, be_ref, z_ref):
    deg = degt_ref[:, 0:1] + degt_ref[:, 1:2] + 1.0
    dinv = lax.rsqrt(jnp.maximum(deg, 1.0))
    t = (s_ref[0] + s_ref[1] - hp_ref[...]) * dinv + b_ref[...]
    r = jnp.maximum(t, 0.0)
    mean = jnp.mean(r, axis=0, keepdims=True)
    cent = r - mean
    var = jnp.mean(cent * cent, axis=0, keepdims=True)
    z_ref[...] = cent * lax.rsqrt(var + BN_EPS) * g_ref[...] + be_ref[...]


_bn_call = pl.pallas_call(
    _bn_body,
    out_shape=jax.ShapeDtypeStruct((N, D), jnp.float32),
)


@jax.jit
def kernel(x, edge_index, W, b, gamma, beta):
    src = edge_index[0]
    dst = edge_index[1].reshape(NW, NCHUNK, CH)
    degp = _deg_call(dst, jnp.zeros((N,), jnp.float32))
    degt = degp.T
    hp = _mm_call(x, W, degt)
    s_parts = _agg_call(hp, src, dst)
    z = _bn_call(
        s_parts, hp, degt,
        b.reshape(1, D), gamma.reshape(1, D), beta.reshape(1, D),
    )
    return z
